# baseline (speedup 1.0000x reference)
import functools

import jax
import jax.numpy as jnp
from jax import lax
from jax.experimental import pallas as pl
from jax.experimental.pallas import tpu as pltpu
from jax.experimental.pallas import tpu_sc as plsc

D = 6144
L = 16
NSL = D // L
K = 8
U = 4
EPS = 1e-5

_GDN = lax.GatherDimensionNumbers(
    offset_dims=(), collapsed_slice_dims=(0,), start_index_map=(0,))


def _lane_perm(x, perm2d):
    return lax.gather(x, perm2d, dimension_numbers=_GDN, slice_sizes=(1,),
                      mode=lax.GatherScatterMode.PROMISE_IN_BOUNDS)


@functools.partial(jax.jit, static_argnums=(0,))
def _sc_embed_ln(n_tokens, ids2d, table, gamma, beta):
    NW = 32
    T = n_tokens // NW
    CH = T // K

    mesh = plsc.VectorSubcoreMesh(core_axis_name="c", subcore_axis_name="s")

    @functools.partial(
        pl.kernel,
        mesh=mesh,
        out_type=jax.ShapeDtypeStruct((n_tokens, D), jnp.float32),
        scratch_types=[
            pltpu.VMEM((CH, K), jnp.int32),
            pltpu.VMEM((K, D), jnp.float32),
            pltpu.VMEM((D,), jnp.float32),
            pltpu.VMEM((D,), jnp.float32),
            pltpu.SemaphoreType.DMA,
        ],
    )
    def k(ids_hbm, table_hbm, gamma_hbm, beta_hbm, out_hbm,
          idx_v, rows_v, gamma_v, beta_v, sem):
        wid = lax.axis_index("s") * 2 + lax.axis_index("c")
        base = wid * T
        pltpu.sync_copy(ids_hbm.at[pl.ds(wid * CH, CH)], idx_v)
        pltpu.sync_copy(gamma_hbm, gamma_v)
        pltpu.sync_copy(beta_hbm, beta_v)

        lane = lax.iota(jnp.int32, L)
        perms = [jnp.reshape(lane ^ st, (L, 1)) for st in (8, 4, 2, 1)]

        def chunk_body(c, carry):
            tok0 = base + c * K
            pltpu.async_copy(table_hbm.at[idx_v.at[c]], rows_v, sem).wait()

            zero = jnp.zeros((L,), jnp.float32)

            def pass1(i, acc):
                out = list(acc)
                for u in range(U):
                    off = (i * U + u) * L
                    for r in range(K):
                        x = rows_v[r, pl.ds(off, L)]
                        out[2 * r] = out[2 * r] + x
                        out[2 * r + 1] = out[2 * r + 1] + x * x
                return tuple(out)

            acc = lax.fori_loop(0, NSL // U, pass1, (zero,) * (2 * K))

            means, rstds = [], []
            for r in range(K):
                s, sq = acc[2 * r], acc[2 * r + 1]
                for p in perms:
                    s = s + _lane_perm(s, p)
                    sq = sq + _lane_perm(sq, p)
                mean_v = s * (1.0 / D)
                a_v = sq * (1.0 / D) - mean_v * mean_v + EPS
                bits = lax.bitcast_convert_type(a_v, jnp.int32)
                bits = 0x5F3759DF - lax.shift_right_logical(bits, 1)
                y = lax.bitcast_convert_type(bits, jnp.float32)
                half = a_v * 0.5
                for _ in range(3):
                    y = y * (1.5 - half * y * y)
                means.append(mean_v)
                rstds.append(y)

            def pass2(i, cc):
                for u in range(U):
                    off = (i * U + u) * L
                    g = gamma_v[pl.ds(off, L)]
                    b = beta_v[pl.ds(off, L)]
                    for r in range(K):
                        x = rows_v[r, pl.ds(off, L)]
                        rows_v[r, pl.ds(off, L)] = (
                            (x - means[r]) * rstds[r] * g + b)
                return cc

            lax.fori_loop(0, NSL // U, pass2, 0)

            pltpu.sync_copy(rows_v, out_hbm.at[pl.ds(tok0, K)])
            return carry

        lax.fori_loop(0, CH, chunk_body, 0)

    return k(ids2d, table, gamma, beta)


def kernel(input_ids, table, gamma, beta):
    b, s = input_ids.shape
    n = b * s
    ids2d = input_ids.reshape(n // K, K).astype(jnp.int32)
    out = _sc_embed_ln(n, ids2d, table, gamma, beta)
    return out.reshape(b, s, D)


# speedup vs baseline: 2.5715x; 2.5715x over previous
import functools

import jax
import jax.numpy as jnp
from jax import lax
from jax.experimental import pallas as pl
from jax.experimental.pallas import tpu as pltpu
from jax.experimental.pallas import tpu_sc as plsc

D = 6144
L = 16
NSL = D // L
K = 4
EPS = 1e-5

_GDN = lax.GatherDimensionNumbers(
    offset_dims=(), collapsed_slice_dims=(0,), start_index_map=(0,))


def _lane_perm(x, perm2d):
    return lax.gather(x, perm2d, dimension_numbers=_GDN, slice_sizes=(1,),
                      mode=lax.GatherScatterMode.PROMISE_IN_BOUNDS)


@functools.partial(jax.jit, static_argnums=(0,))
def _sc_embed_ln(n_tokens, ids2d, table, gamma, beta):
    NW = 32
    T = n_tokens // NW
    CH = T // K
    HC = CH // 2

    mesh = plsc.VectorSubcoreMesh(core_axis_name="c", subcore_axis_name="s")

    @functools.partial(
        pl.kernel,
        mesh=mesh,
        out_type=jax.ShapeDtypeStruct((n_tokens, D), jnp.float32),
        scratch_types=[
            pltpu.VMEM((CH, K), jnp.int32),
            pltpu.VMEM((K, D), jnp.float32),
            pltpu.VMEM((K, D), jnp.float32),
            pltpu.VMEM((K, D), jnp.float32),
            pltpu.VMEM((K, D), jnp.float32),
            pltpu.VMEM((D,), jnp.float32),
            pltpu.VMEM((D,), jnp.float32),
            pltpu.SemaphoreType.DMA,
            pltpu.SemaphoreType.DMA,
            pltpu.SemaphoreType.DMA,
            pltpu.SemaphoreType.DMA,
        ],
    )
    def k(ids_hbm, table_hbm, gamma_hbm, beta_hbm, out_hbm,
          idx_v, ibuf0, ibuf1, obuf0, obuf1, gamma_v, beta_v,
          gsem0, gsem1, osem0, osem1):
        wid = lax.axis_index("s") * 2 + lax.axis_index("c")
        base = wid * T
        pltpu.sync_copy(ids_hbm.at[pl.ds(wid * CH, CH)], idx_v)
        pltpu.sync_copy(gamma_hbm, gamma_v)
        pltpu.sync_copy(beta_hbm, beta_v)

        ibufs, obufs = (ibuf0, ibuf1), (obuf0, obuf1)
        gsems, osems = (gsem0, gsem1), (osem0, osem1)

        lane = lax.iota(jnp.int32, L)
        perms = [jnp.reshape(lane ^ st, (L, 1)) for st in (8, 4, 2, 1)]

        pltpu.async_copy(table_hbm.at[idx_v.at[0]], ibuf0, gsem0)
        pltpu.async_copy(table_hbm.at[idx_v.at[1]], ibuf1, gsem1)

        def pair_body(cc, carry):
            for par in (0, 1):
                c = 2 * cc + par
                ibuf, obuf = ibufs[par], obufs[par]
                gsem, osem = gsems[par], osems[par]
                tok0 = base + c * K

                pltpu.make_async_copy(
                    table_hbm.at[idx_v.at[c]], ibuf, gsem).wait()

                zero = jnp.zeros((L,), jnp.float32)

                def pass1(i, acc):
                    out = list(acc)
                    off = i * L
                    for r in range(K):
                        x = ibuf[r, pl.ds(off, L)]
                        out[2 * r] = out[2 * r] + x
                        out[2 * r + 1] = out[2 * r + 1] + x * x
                    return tuple(out)

                acc = lax.fori_loop(0, NSL, pass1, (zero,) * (2 * K))

                means, rstds = [], []
                for r in range(K):
                    s, sq = acc[2 * r], acc[2 * r + 1]
                    for p in perms:
                        s = s + _lane_perm(s, p)
                        sq = sq + _lane_perm(sq, p)
                    mean_v = s * (1.0 / D)
                    a_v = sq * (1.0 / D) - mean_v * mean_v + EPS
                    bits = lax.bitcast_convert_type(a_v, jnp.int32)
                    bits = 0x5F3759DF - lax.shift_right_logical(bits, 1)
                    y = lax.bitcast_convert_type(bits, jnp.float32)
                    half = a_v * 0.5
                    for _ in range(3):
                        y = y * (1.5 - half * y * y)
                    means.append(mean_v)
                    rstds.append(y)

                @pl.when(cc >= 1)
                def _():
                    pltpu.make_async_copy(
                        obuf, out_hbm.at[pl.ds(base, K)], osem).wait()

                def pass2(i, cc2):
                    off = i * L
                    g = gamma_v[pl.ds(off, L)]
                    b = beta_v[pl.ds(off, L)]
                    for r in range(K):
                        x = ibuf[r, pl.ds(off, L)]
                        obuf[r, pl.ds(off, L)] = (
                            (x - means[r]) * rstds[r] * g + b)
                    return cc2

                lax.fori_loop(0, NSL, pass2, 0)

                pltpu.async_copy(obuf, out_hbm.at[pl.ds(tok0, K)], osem)

                @pl.when(cc < HC - 1)
                def _():
                    pltpu.async_copy(
                        table_hbm.at[idx_v.at[c + 2]], ibuf, gsem)

            return carry

        lax.fori_loop(0, HC, pair_body, 0)

        pltpu.make_async_copy(obuf0, out_hbm.at[pl.ds(base, K)], osem0).wait()
        pltpu.make_async_copy(obuf1, out_hbm.at[pl.ds(base, K)], osem1).wait()

    return k(ids2d, table, gamma, beta)


def kernel(input_ids, table, gamma, beta):
    b, s = input_ids.shape
    n = b * s
    ids2d = input_ids.reshape(n // K, K).astype(jnp.int32)
    out = _sc_embed_ln(n, ids2d, table, gamma, beta)
    return out.reshape(b, s, D)


# speedup vs baseline: 3.7076x; 1.4418x over previous
import functools

import jax
import jax.numpy as jnp
from jax import lax
from jax.experimental import pallas as pl
from jax.experimental.pallas import tpu as pltpu
from jax.experimental.pallas import tpu_sc as plsc

D = 6144
L = 16
NSL = D // L
K = 4
EPS = 1e-5

_GDN = lax.GatherDimensionNumbers(
    offset_dims=(), collapsed_slice_dims=(0,), start_index_map=(0,))


def _lane_perm(x, perm2d):
    return lax.gather(x, perm2d, dimension_numbers=_GDN, slice_sizes=(1,),
                      mode=lax.GatherScatterMode.PROMISE_IN_BOUNDS)


@functools.partial(jax.jit, static_argnums=(0,))
def _sc_embed_ln(n_tokens, ids2d, table, gamma, beta):
    NW = 32
    T = n_tokens // NW
    CH = T // K
    HC = CH // 2

    mesh = plsc.VectorSubcoreMesh(core_axis_name="c", subcore_axis_name="s")

    @functools.partial(
        pl.kernel,
        mesh=mesh,
        out_type=jax.ShapeDtypeStruct((n_tokens, D), jnp.float32),
        scratch_types=[
            pltpu.VMEM((CH, K), jnp.int32),
            pltpu.VMEM((K, D), jnp.float32),
            pltpu.VMEM((K, D), jnp.float32),
            pltpu.VMEM((K, D), jnp.float32),
            pltpu.VMEM((K, D), jnp.float32),
            pltpu.VMEM((D,), jnp.float32),
            pltpu.VMEM((D,), jnp.float32),
            pltpu.SemaphoreType.DMA,
            pltpu.SemaphoreType.DMA,
            pltpu.SemaphoreType.DMA,
            pltpu.SemaphoreType.DMA,
        ],
    )
    def k(ids_hbm, table_hbm, gamma_hbm, beta_hbm, out_hbm,
          idx_v, ibuf0, ibuf1, obuf0, obuf1, gamma_v, beta_v,
          gsem0, gsem1, osem0, osem1):
        wid = lax.axis_index("s") * 2 + lax.axis_index("c")
        base = wid * T
        pltpu.sync_copy(ids_hbm.at[pl.ds(wid * CH, CH)], idx_v)
        pltpu.sync_copy(gamma_hbm, gamma_v)
        pltpu.sync_copy(beta_hbm, beta_v)

        ibufs, obufs = (ibuf0, ibuf1), (obuf0, obuf1)
        gsems, osems = (gsem0, gsem1), (osem0, osem1)

        lane = lax.iota(jnp.int32, L)
        perms = [jnp.reshape(lane ^ st, (L, 1)) for st in (8, 4, 2, 1)]

        pltpu.async_copy(table_hbm.at[idx_v.at[0]], ibuf0, gsem0)
        pltpu.async_copy(table_hbm.at[idx_v.at[1]], ibuf1, gsem1)

        def pair_body(cc, carry):
            for par in (0, 1):
                c = 2 * cc + par
                ibuf, obuf = ibufs[par], obufs[par]
                gsem, osem = gsems[par], osems[par]
                tok0 = base + c * K

                pltpu.make_async_copy(
                    table_hbm.at[idx_v.at[c]], ibuf, gsem).wait()

                zero = jnp.zeros((L,), jnp.float32)

                @plsc.parallel_loop(0, NSL, unroll=4, carry=(zero,) * (2 * K))
                def acc(i, acc_in):
                    out = list(acc_in)
                    off = i * L
                    for r in range(K):
                        x = ibuf[r, pl.ds(off, L)]
                        out[2 * r] = out[2 * r] + x
                        out[2 * r + 1] = out[2 * r + 1] + x * x
                    return tuple(out)

                scales, shifts = [], []
                for r in range(K):
                    s, sq = acc[2 * r], acc[2 * r + 1]
                    for p in perms:
                        s = s + _lane_perm(s, p)
                        sq = sq + _lane_perm(sq, p)
                    mean_v = s * (1.0 / D)
                    a_v = sq * (1.0 / D) - mean_v * mean_v + EPS
                    bits = lax.bitcast_convert_type(a_v, jnp.int32)
                    bits = 0x5F3759DF - lax.shift_right_logical(bits, 1)
                    y = lax.bitcast_convert_type(bits, jnp.float32)
                    half = a_v * 0.5
                    for _ in range(3):
                        y = y * (1.5 - half * y * y)
                    scales.append(y)
                    shifts.append(-mean_v * y)

                @pl.when(cc >= 1)
                def _():
                    pltpu.make_async_copy(
                        obuf, out_hbm.at[pl.ds(base, K)], osem).wait()

                @plsc.parallel_loop(0, NSL, unroll=4)
                def _(i):
                    off = i * L
                    g = gamma_v[pl.ds(off, L)]
                    b = beta_v[pl.ds(off, L)]
                    for r in range(K):
                        x = ibuf[r, pl.ds(off, L)]
                        t = x * scales[r] + shifts[r]
                        obuf[r, pl.ds(off, L)] = t * g + b

                pltpu.async_copy(obuf, out_hbm.at[pl.ds(tok0, K)], osem)

                @pl.when(cc < HC - 1)
                def _():
                    pltpu.async_copy(
                        table_hbm.at[idx_v.at[c + 2]], ibuf, gsem)

            return carry

        lax.fori_loop(0, HC, pair_body, 0)

        pltpu.make_async_copy(obuf0, out_hbm.at[pl.ds(base, K)], osem0).wait()
        pltpu.make_async_copy(obuf1, out_hbm.at[pl.ds(base, K)], osem1).wait()

    return k(ids2d, table, gamma, beta)


def kernel(input_ids, table, gamma, beta):
    b, s = input_ids.shape
    n = b * s
    ids2d = input_ids.reshape(n // K, K).astype(jnp.int32)
    out = _sc_embed_ln(n, ids2d, table, gamma, beta)
    return out.reshape(b, s, D)
